# baseline (speedup 1.0000x reference)
import functools

import jax
import jax.numpy as jnp
import numpy as np
from jax import lax
from jax.experimental import pallas as pl
from jax.experimental.pallas import tpu as pltpu
from jax.experimental.pallas import tpu_sc as plsc

VOCAB = 100000
DIM = 64
N_REFS = 20
BATCH = 16384
MARGIN = 0.5
EPS = 1e-6

NC = 2
NS = 16
NW = NC * NS
CHUNK = BATCH // NW
G = 32
NG = CHUNK // G


REFS_PAD = 32
IDXC = 128
NQ = CHUNK // IDXC
GR = G * N_REFS
NS_SUB = GR // IDXC
NFLAT = CHUNK * N_REFS


def _sc_embed_body(i1, o1, n1, refs_hbm, win_hbm, wout_hbm,
                   isum, osum, nsum,
                   words_v, pos_v, flat_v, rows0_v, rows1_v,
                   out0_v, out1_v, sem_r, sem_g0, sem_g1, sem_o0, sem_o1):
  wid = lax.axis_index("s") * NC + lax.axis_index("c")
  base = wid * CHUNK
  rows = (rows0_v, rows1_v)
  outb = (out0_v, out1_v)
  sem_g = (sem_g0, sem_g1)
  sem_o = (sem_o0, sem_o1)

  for word1, table, out in ((i1, win_hbm, isum),
                            (o1, wout_hbm, osum),
                            (n1, wout_hbm, nsum)):
    pltpu.sync_copy(word1.at[pl.ds(wid * CHUNK, CHUNK)], words_v)

    def pos_body(v, _):
      k = v * 16 + lax.iota(jnp.int32, 16)
      row = lax.div(k, jnp.int32(N_REFS))
      col = k - row * N_REFS
      w = plsc.load_gather(words_v, [row])
      pos_v[pl.ds(v * 16, 16)] = w * N_REFS + col
      return 0

    lax.fori_loop(0, NFLAT // 16, pos_body, 0)

    def ref_copy(q):
      return pltpu.make_async_copy(
          refs_hbm.at[pos_v.at[pl.ds(q * IDXC, IDXC)]],
          flat_v.at[pl.ds(q * IDXC, IDXC)], sem_r)

    lax.fori_loop(0, NFLAT // IDXC, lambda q, _: (ref_copy(q).start(), 0)[1], 0)
    lax.fori_loop(0, NFLAT // IDXC, lambda q, _: (ref_copy(q).wait(), 0)[1], 0)

    def emb_copies(g, b):
      return [
          pltpu.make_async_copy(
              table.at[flat_v.at[pl.ds(g * GR + s * IDXC, IDXC)]],
              rows[b].at[pl.ds(s * IDXC, IDXC)], sem_g[b])
          for s in range(NS_SUB)
      ]

    def fire(g, b):
      for d in emb_copies(g, b):
        d.start()

    def drain(g, b):
      for d in emb_copies(g, b):
        d.wait()

    def out_copy(g, b):
      return pltpu.make_async_copy(
          outb[b], out.at[pl.ds((base + g * G) // 2, G // 2)], sem_o[b])

    fire(0, 0)

    def two_groups(h, _):
      for b in range(2):
        g = 2 * h + b

        @pl.when(g + 1 < NG)
        def _():
          fire(g + 1, 1 - b)

        drain(g, b)

        @pl.when(g >= 2)
        def _():
          out_copy(g - 2, b).wait()

        def item_body(i, _):
          r0 = i * N_REFS
          acc0 = [rows[b][r0, pl.ds(c * 16, 16)] for c in range(DIM // 16)]
          acc1 = [rows[b][r0 + 1, pl.ds(c * 16, 16)]
                  for c in range(DIM // 16)]
          for j in range(2, N_REFS, 2):
            for c in range(DIM // 16):
              acc0[c] = acc0[c] + rows[b][r0 + j, pl.ds(c * 16, 16)]
              acc1[c] = acc1[c] + rows[b][r0 + j + 1, pl.ds(c * 16, 16)]
          half = lax.shift_right_logical(i, 1)
          off = (i & 1) * DIM
          for c in range(DIM // 16):
            outb[b][half, pl.ds(off + c * 16, 16)] = acc0[c] + acc1[c]
          return 0

        lax.fori_loop(0, G, item_body, 0)
        out_copy(g, b).start()
      return 0

    lax.fori_loop(0, NG // 2, two_groups, 0)
    out_copy(NG - 2, 0).wait()
    out_copy(NG - 1, 1).wait()


def _sc_embed(iword, oword, nword, refs, w_in, w_out):
  refs = refs.reshape(-1)
  mesh = plsc.VectorSubcoreMesh(core_axis_name="c", subcore_axis_name="s")
  f = pl.kernel(
      _sc_embed_body,
      out_type=[jax.ShapeDtypeStruct((BATCH // 2, 2 * DIM), jnp.float32)] * 3,
      mesh=mesh,
      compiler_params=pltpu.CompilerParams(use_tc_tiling_on_sc=False,
                                           needs_layout_passes=False),
      scratch_types=[
          pltpu.VMEM((CHUNK,), jnp.int32),
          pltpu.VMEM((NFLAT,), jnp.int32),
          pltpu.VMEM((NFLAT,), jnp.int32),
          pltpu.VMEM((GR, DIM), jnp.float32),
          pltpu.VMEM((GR, DIM), jnp.float32),
          pltpu.VMEM((G // 2, 2 * DIM), jnp.float32),
          pltpu.VMEM((G // 2, 2 * DIM), jnp.float32),
          pltpu.SemaphoreType.DMA,
          pltpu.SemaphoreType.DMA,
          pltpu.SemaphoreType.DMA,
          pltpu.SemaphoreType.DMA,
          pltpu.SemaphoreType.DMA,
      ],
  )
  return f(iword, oword, nword, refs, w_in, w_out)


def _log_sigmoid(x):
  return jnp.minimum(x, 0.0) - jnp.log1p(jnp.exp(-jnp.abs(x)))


def _half_loss(iv, ov, nv):
  dio = jnp.sum(iv * ov, axis=1)
  din = jnp.sum(iv * nv, axis=1)
  ni = jnp.sqrt(jnp.sum(iv * iv, axis=1))
  no = jnp.sqrt(jnp.sum(ov * ov, axis=1))
  nn = jnp.sqrt(jnp.sum(nv * nv, axis=1))
  cos_io = dio / (jnp.maximum(ni, EPS) * jnp.maximum(no, EPS))
  cos_in = din / (jnp.maximum(ni, EPS) * jnp.maximum(nn, EPS))
  x_p = MARGIN * cos_io
  x_n = MARGIN * cos_in
  pos_angle = jnp.arctan2(jnp.sqrt(jnp.maximum(1.0 - x_p * x_p, 0.0)), x_p)
  neg_angle = jnp.arctan2(jnp.sqrt(jnp.maximum(1.0 - x_n * x_n, 0.0)), x_n)
  pos_rad = ni * no
  neg_rad = ni * nn
  inv_pi = np.float32(1.0 / np.pi)
  oloss = _log_sigmoid(-pos_angle * pos_rad * inv_pi)
  nloss = _log_sigmoid(neg_angle * neg_rad * inv_pi)
  return jnp.sum(oloss + nloss)


def _loss_body(is_ref, os_ref, ns_ref, out_ref):
  inv = np.float32(1.0 / N_REFS)
  iv = is_ref[...] * inv
  ov = os_ref[...] * inv
  nv = ns_ref[...] * inv
  tot = (_half_loss(iv[:, :DIM], ov[:, :DIM], nv[:, :DIM]) +
         _half_loss(iv[:, DIM:], ov[:, DIM:], nv[:, DIM:]))
  part = -tot * np.float32(1.0 / BATCH)
  pid = pl.program_id(0)

  @pl.when(pid == 0)
  def _():
    out_ref[0, 0] = part

  @pl.when(pid != 0)
  def _():
    out_ref[0, 0] += part


LOSS_BLK = 4096


def _loss(isum, osum, nsum):
  nblk = BATCH // 2 // LOSS_BLK
  spec = pl.BlockSpec((LOSS_BLK, 2 * DIM), lambda i: (i, 0))
  f = pl.pallas_call(
      _loss_body,
      grid=(nblk,),
      in_specs=[spec, spec, spec],
      out_shape=jax.ShapeDtypeStruct((1, 1), jnp.float32),
      out_specs=pl.BlockSpec(memory_space=pltpu.SMEM),
  )
  return f(isum, osum, nsum)[0, 0]


def kernel(iword, oword, nword, refs, W_in, W_out):
  iword = iword.astype(jnp.int32)
  oword = oword.astype(jnp.int32)
  nword = nword.astype(jnp.int32)
  refs = refs.astype(jnp.int32)
  isum, osum, nsum = _sc_embed(iword, oword, nword, refs, W_in, W_out)
  return _loss(isum, osum, nsum)


# speedup vs baseline: 1.1199x; 1.1199x over previous
import functools

import jax
import jax.numpy as jnp
import numpy as np
from jax import lax
from jax.experimental import pallas as pl
from jax.experimental.pallas import tpu as pltpu
from jax.experimental.pallas import tpu_sc as plsc

VOCAB = 100000
DIM = 64
N_REFS = 20
BATCH = 16384
MARGIN = 0.5
EPS = 1e-6

NC = 2
NS = 16
NW = NC * NS
CHUNK = BATCH // NW
G = 32
NG = CHUNK // G


REFS_PAD = 32
IDXC = 128
NQ = CHUNK // IDXC
GR = G * N_REFS
NS_SUB = GR // IDXC
NFLAT = CHUNK * N_REFS


def _sc_embed_body(words_hbm, refs_hbm, win_hbm, wout_hbm,
                   isum, osum, nsum,
                   words_v, pos_v, flat_v, rows0_v, rows1_v,
                   out0_v, out1_v, sem_r, sem_g0, sem_g1, sem_o0, sem_o1):
  wid = lax.axis_index("s") * NC + lax.axis_index("c")
  base = wid * CHUNK
  rows = (rows0_v, rows1_v)
  outb = (out0_v, out1_v)
  sem_g = (sem_g0, sem_g1)
  sem_o = (sem_o0, sem_o1)

  for a, (table, out) in enumerate(((win_hbm, isum),
                                    (wout_hbm, osum),
                                    (wout_hbm, nsum))):
    pltpu.sync_copy(words_hbm.at[pl.ds(a * BATCH + wid * CHUNK, CHUNK)],
                    words_v)

    def pos_body(v, _):
      k = v * 16 + lax.iota(jnp.int32, 16)
      row = lax.div(k, jnp.int32(N_REFS))
      col = k - row * N_REFS
      w = plsc.load_gather(words_v, [row])
      pos_v[pl.ds(v * 16, 16)] = w * N_REFS + col
      return 0

    lax.fori_loop(0, NFLAT // 16, pos_body, 0)

    def ref_copy(q):
      return pltpu.make_async_copy(
          refs_hbm.at[pos_v.at[pl.ds(q * IDXC, IDXC)]],
          flat_v.at[pl.ds(q * IDXC, IDXC)], sem_r)

    lax.fori_loop(0, NFLAT // IDXC, lambda q, _: (ref_copy(q).start(), 0)[1], 0)
    lax.fori_loop(0, NFLAT // IDXC, lambda q, _: (ref_copy(q).wait(), 0)[1], 0)

    def emb_copies(g, b):
      return [
          pltpu.make_async_copy(
              table.at[flat_v.at[pl.ds(g * GR + s * IDXC, IDXC)]],
              rows[b].at[pl.ds(s * IDXC, IDXC)], sem_g[b])
          for s in range(NS_SUB)
      ]

    def fire(g, b):
      for d in emb_copies(g, b):
        d.start()

    def drain(g, b):
      for d in emb_copies(g, b):
        d.wait()

    def out_copy(g, b):
      return pltpu.make_async_copy(
          outb[b], out.at[pl.ds((base + g * G) // 2, G // 2)], sem_o[b])

    fire(0, 0)

    def two_groups(h, _):
      for b in range(2):
        g = 2 * h + b

        @pl.when(g + 1 < NG)
        def _():
          fire(g + 1, 1 - b)

        drain(g, b)

        @pl.when(g >= 2)
        def _():
          out_copy(g - 2, b).wait()

        def item_body(i, _):
          r0 = i * N_REFS
          acc0 = [rows[b][r0, pl.ds(c * 16, 16)] for c in range(DIM // 16)]
          acc1 = [rows[b][r0 + 1, pl.ds(c * 16, 16)]
                  for c in range(DIM // 16)]
          for j in range(2, N_REFS, 2):
            for c in range(DIM // 16):
              acc0[c] = acc0[c] + rows[b][r0 + j, pl.ds(c * 16, 16)]
              acc1[c] = acc1[c] + rows[b][r0 + j + 1, pl.ds(c * 16, 16)]
          half = lax.shift_right_logical(i, 1)
          off = (i & 1) * DIM
          for c in range(DIM // 16):
            outb[b][half, pl.ds(off + c * 16, 16)] = acc0[c] + acc1[c]
          return 0

        lax.fori_loop(0, G, item_body, 0)
        out_copy(g, b).start()
      return 0

    lax.fori_loop(0, NG // 2, two_groups, 0)
    out_copy(NG - 2, 0).wait()
    out_copy(NG - 1, 1).wait()


def _sc_embed(iword, oword, nword, refs, w_in, w_out):
  words = jnp.concatenate([iword, oword, nword])
  refs = refs.reshape(-1)
  mesh = plsc.VectorSubcoreMesh(core_axis_name="c", subcore_axis_name="s")
  f = pl.kernel(
      _sc_embed_body,
      out_type=[jax.ShapeDtypeStruct((BATCH // 2, 2 * DIM), jnp.float32)] * 3,
      mesh=mesh,
      compiler_params=pltpu.CompilerParams(use_tc_tiling_on_sc=False,
                                           needs_layout_passes=False),
      scratch_types=[
          pltpu.VMEM((CHUNK,), jnp.int32),
          pltpu.VMEM((NFLAT,), jnp.int32),
          pltpu.VMEM((NFLAT,), jnp.int32),
          pltpu.VMEM((GR, DIM), jnp.float32),
          pltpu.VMEM((GR, DIM), jnp.float32),
          pltpu.VMEM((G // 2, 2 * DIM), jnp.float32),
          pltpu.VMEM((G // 2, 2 * DIM), jnp.float32),
          pltpu.SemaphoreType.DMA,
          pltpu.SemaphoreType.DMA,
          pltpu.SemaphoreType.DMA,
          pltpu.SemaphoreType.DMA,
          pltpu.SemaphoreType.DMA,
      ],
  )
  return f(words, refs, w_in, w_out)


def _log_sigmoid(x):
  return jnp.minimum(x, 0.0) - jnp.log1p(jnp.exp(-jnp.abs(x)))


_ASIN_C = [np.float32(c) for c in (
    1.0, 1 / 6, 3 / 40, 15 / 336, 105 / 3456, 945 / 42240,
    10395 / 599040, 135135 / 9676800)]


def _asin_small(x):
  u = x * x
  s = _ASIN_C[-1]
  for c in _ASIN_C[-2::-1]:
    s = s * u + c
  return x * s


def _half_loss(iv, ov, nv):
  dio = jnp.sum(iv * ov, axis=1)
  din = jnp.sum(iv * nv, axis=1)
  ni = jnp.sqrt(jnp.sum(iv * iv, axis=1))
  no = jnp.sqrt(jnp.sum(ov * ov, axis=1))
  nn = jnp.sqrt(jnp.sum(nv * nv, axis=1))
  cos_io = dio / (jnp.maximum(ni, EPS) * jnp.maximum(no, EPS))
  cos_in = din / (jnp.maximum(ni, EPS) * jnp.maximum(nn, EPS))
  x_p = MARGIN * cos_io
  x_n = MARGIN * cos_in
  pos_angle = np.float32(np.pi / 2) - _asin_small(x_p)
  neg_angle = np.float32(np.pi / 2) - _asin_small(x_n)
  pos_rad = ni * no
  neg_rad = ni * nn
  inv_pi = np.float32(1.0 / np.pi)
  oloss = _log_sigmoid(-pos_angle * pos_rad * inv_pi)
  nloss = _log_sigmoid(neg_angle * neg_rad * inv_pi)
  return jnp.sum(oloss + nloss)


def _loss_body(is_ref, os_ref, ns_ref, out_ref):
  inv = np.float32(1.0 / N_REFS)
  iv = is_ref[...] * inv
  ov = os_ref[...] * inv
  nv = ns_ref[...] * inv
  tot = (_half_loss(iv[:, :DIM], ov[:, :DIM], nv[:, :DIM]) +
         _half_loss(iv[:, DIM:], ov[:, DIM:], nv[:, DIM:]))
  part = -tot * np.float32(1.0 / BATCH)
  pid = pl.program_id(0)

  @pl.when(pid == 0)
  def _():
    out_ref[0, 0] = part

  @pl.when(pid != 0)
  def _():
    out_ref[0, 0] += part


LOSS_BLK = 1024


def _loss(isum, osum, nsum):
  nblk = BATCH // 2 // LOSS_BLK
  spec = pl.BlockSpec((LOSS_BLK, 2 * DIM), lambda i: (i, 0))
  f = pl.pallas_call(
      _loss_body,
      grid=(nblk,),
      in_specs=[spec, spec, spec],
      out_shape=jax.ShapeDtypeStruct((1, 1), jnp.float32),
      out_specs=pl.BlockSpec(memory_space=pltpu.SMEM),
  )
  return f(isum, osum, nsum)[0, 0]


def kernel(iword, oword, nword, refs, W_in, W_out):
  iword = iword.astype(jnp.int32)
  oword = oword.astype(jnp.int32)
  nword = nword.astype(jnp.int32)
  refs = refs.astype(jnp.int32)
  isum, osum, nsum = _sc_embed(iword, oword, nword, refs, W_in, W_out)
  return _loss(isum, osum, nsum)


# speedup vs baseline: 1.1369x; 1.0152x over previous
import functools

import jax
import jax.numpy as jnp
import numpy as np
from jax import lax
from jax.experimental import pallas as pl
from jax.experimental.pallas import tpu as pltpu
from jax.experimental.pallas import tpu_sc as plsc

VOCAB = 100000
DIM = 64
N_REFS = 20
BATCH = 16384
MARGIN = 0.5
EPS = 1e-6

NC = 2
NS = 16
NW = NC * NS
CHUNK = BATCH // NW
G = 32
NG = CHUNK // G


REFS_PAD = 32
IDXC = 128
NQ = CHUNK // IDXC
GR = G * N_REFS
NS_SUB = GR // IDXC
NFLAT = CHUNK * N_REFS


PREFETCH_G = 4


def _sc_embed_body(words_hbm, refs_hbm, win_hbm, wout_hbm,
                   isum, osum, nsum,
                   words_v, pos_v, flat0_v, flat1_v, rows0_v, rows1_v,
                   out0_v, out1_v, sem_r, sem_g0, sem_g1, sem_o0, sem_o1):
  wid = lax.axis_index("s") * NC + lax.axis_index("c")
  base = wid * CHUNK
  rows = (rows0_v, rows1_v)
  flats = (flat0_v, flat1_v)
  outb = (out0_v, out1_v)
  sem_g = (sem_g0, sem_g1)
  sem_o = (sem_o0, sem_o1)

  def stage_ids(a, fa):
    pltpu.sync_copy(words_hbm.at[pl.ds(a * BATCH + wid * CHUNK, CHUNK)],
                    words_v)

    def pos_body(v, _):
      k = v * 16 + lax.iota(jnp.int32, 16)
      row = lax.div(k, jnp.int32(N_REFS))
      col = k - row * N_REFS
      w = plsc.load_gather(words_v, [row])
      pos_v[pl.ds(v * 16, 16)] = w * N_REFS + col
      return 0

    lax.fori_loop(0, NFLAT // 16, pos_body, 0)
    lax.fori_loop(0, NFLAT // IDXC,
                  lambda q, _: (ref_copy(q, fa).start(), 0)[1], 0)

  def ref_copy(q, fa):
    return pltpu.make_async_copy(
        refs_hbm.at[pos_v.at[pl.ds(q * IDXC, IDXC)]],
        flats[fa].at[pl.ds(q * IDXC, IDXC)], sem_r)

  def drain_ids(fa):
    lax.fori_loop(0, NFLAT // IDXC,
                  lambda q, _: (ref_copy(q, fa).wait(), 0)[1], 0)

  stage_ids(0, 0)
  drain_ids(0)

  for a, (table, out) in enumerate(((win_hbm, isum),
                                    (wout_hbm, osum),
                                    (wout_hbm, nsum))):
    fa = a % 2

    def emb_copies(g, b):
      return [
          pltpu.make_async_copy(
              table.at[flats[fa].at[pl.ds(g * GR + s * IDXC, IDXC)]],
              rows[b].at[pl.ds(s * IDXC, IDXC)], sem_g[b])
          for s in range(NS_SUB)
      ]

    def fire(g, b):
      for d in emb_copies(g, b):
        d.start()

    def drain(g, b):
      for d in emb_copies(g, b):
        d.wait()

    def out_copy(g, b):
      return pltpu.make_async_copy(
          outb[b], out.at[pl.ds((base + g * G) // 2, G // 2)], sem_o[b])

    if a > 0:
      drain_ids(fa)
    fire(0, 0)

    def two_groups(h, _):
      for b in range(2):
        g = 2 * h + b

        @pl.when(g + 1 < NG)
        def _():
          fire(g + 1, 1 - b)

        if a < 2:
          @pl.when(g == PREFETCH_G)
          def _():
            stage_ids(a + 1, 1 - fa)

        drain(g, b)

        @pl.when(g >= 2)
        def _():
          out_copy(g - 2, b).wait()

        def item_body(i, _):
          r0 = i * N_REFS
          acc0 = [rows[b][r0, pl.ds(c * 16, 16)] for c in range(DIM // 16)]
          acc1 = [rows[b][r0 + 1, pl.ds(c * 16, 16)]
                  for c in range(DIM // 16)]
          for j in range(2, N_REFS, 2):
            for c in range(DIM // 16):
              acc0[c] = acc0[c] + rows[b][r0 + j, pl.ds(c * 16, 16)]
              acc1[c] = acc1[c] + rows[b][r0 + j + 1, pl.ds(c * 16, 16)]
          half = lax.shift_right_logical(i, 1)
          off = (i & 1) * DIM
          for c in range(DIM // 16):
            outb[b][half, pl.ds(off + c * 16, 16)] = acc0[c] + acc1[c]
          return 0

        lax.fori_loop(0, G, item_body, 0)
        out_copy(g, b).start()
      return 0

    lax.fori_loop(0, NG // 2, two_groups, 0)
    out_copy(NG - 2, 0).wait()
    out_copy(NG - 1, 1).wait()


def _sc_embed(iword, oword, nword, refs, w_in, w_out):
  words = jnp.concatenate([iword, oword, nword])
  refs = refs.reshape(-1)
  mesh = plsc.VectorSubcoreMesh(core_axis_name="c", subcore_axis_name="s")
  f = pl.kernel(
      _sc_embed_body,
      out_type=[jax.ShapeDtypeStruct((BATCH // 2, 2 * DIM), jnp.float32)] * 3,
      mesh=mesh,
      compiler_params=pltpu.CompilerParams(use_tc_tiling_on_sc=False,
                                           needs_layout_passes=False),
      scratch_types=[
          pltpu.VMEM((CHUNK,), jnp.int32),
          pltpu.VMEM((NFLAT,), jnp.int32),
          pltpu.VMEM((NFLAT,), jnp.int32),
          pltpu.VMEM((NFLAT,), jnp.int32),
          pltpu.VMEM((GR, DIM), jnp.float32),
          pltpu.VMEM((GR, DIM), jnp.float32),
          pltpu.VMEM((G // 2, 2 * DIM), jnp.float32),
          pltpu.VMEM((G // 2, 2 * DIM), jnp.float32),
          pltpu.SemaphoreType.DMA,
          pltpu.SemaphoreType.DMA,
          pltpu.SemaphoreType.DMA,
          pltpu.SemaphoreType.DMA,
          pltpu.SemaphoreType.DMA,
      ],
  )
  return f(words, refs, w_in, w_out)


def _log_sigmoid(x):
  return jnp.minimum(x, 0.0) - jnp.log1p(jnp.exp(-jnp.abs(x)))


_ASIN_C = [np.float32(c) for c in (
    1.0, 1 / 6, 3 / 40, 15 / 336, 105 / 3456, 945 / 42240,
    10395 / 599040, 135135 / 9676800)]


def _asin_small(x):
  u = x * x
  s = _ASIN_C[-1]
  for c in _ASIN_C[-2::-1]:
    s = s * u + c
  return x * s


def _half_loss(iv, ov, nv):
  dio = jnp.sum(iv * ov, axis=1)
  din = jnp.sum(iv * nv, axis=1)
  ni = jnp.sqrt(jnp.sum(iv * iv, axis=1))
  no = jnp.sqrt(jnp.sum(ov * ov, axis=1))
  nn = jnp.sqrt(jnp.sum(nv * nv, axis=1))
  cos_io = dio / (jnp.maximum(ni, EPS) * jnp.maximum(no, EPS))
  cos_in = din / (jnp.maximum(ni, EPS) * jnp.maximum(nn, EPS))
  x_p = MARGIN * cos_io
  x_n = MARGIN * cos_in
  pos_angle = np.float32(np.pi / 2) - _asin_small(x_p)
  neg_angle = np.float32(np.pi / 2) - _asin_small(x_n)
  pos_rad = ni * no
  neg_rad = ni * nn
  inv_pi = np.float32(1.0 / np.pi)
  oloss = _log_sigmoid(-pos_angle * pos_rad * inv_pi)
  nloss = _log_sigmoid(neg_angle * neg_rad * inv_pi)
  return jnp.sum(oloss + nloss)


def _loss_body(is_ref, os_ref, ns_ref, out_ref):
  inv = np.float32(1.0 / N_REFS)
  iv = is_ref[...] * inv
  ov = os_ref[...] * inv
  nv = ns_ref[...] * inv
  tot = (_half_loss(iv[:, :DIM], ov[:, :DIM], nv[:, :DIM]) +
         _half_loss(iv[:, DIM:], ov[:, DIM:], nv[:, DIM:]))
  part = -tot * np.float32(1.0 / BATCH)
  pid = pl.program_id(0)

  @pl.when(pid == 0)
  def _():
    out_ref[0, 0] = part

  @pl.when(pid != 0)
  def _():
    out_ref[0, 0] += part


LOSS_BLK = 1024


def _loss(isum, osum, nsum):
  nblk = BATCH // 2 // LOSS_BLK
  spec = pl.BlockSpec((LOSS_BLK, 2 * DIM), lambda i: (i, 0))
  f = pl.pallas_call(
      _loss_body,
      grid=(nblk,),
      in_specs=[spec, spec, spec],
      out_shape=jax.ShapeDtypeStruct((1, 1), jnp.float32),
      out_specs=pl.BlockSpec(memory_space=pltpu.SMEM),
  )
  return f(isum, osum, nsum)[0, 0]


def kernel(iword, oword, nword, refs, W_in, W_out):
  iword = iword.astype(jnp.int32)
  oword = oword.astype(jnp.int32)
  nword = nword.astype(jnp.int32)
  refs = refs.astype(jnp.int32)
  isum, osum, nsum = _sc_embed(iword, oword, nword, refs, W_in, W_out)
  return _loss(isum, osum, nsum)


# speedup vs baseline: 1.1405x; 1.0031x over previous
import jax
import jax.numpy as jnp
import numpy as np
from jax import lax
from jax.experimental import pallas as pl
from jax.experimental.pallas import tpu as pltpu
from jax.experimental.pallas import tpu_sc as plsc

VOCAB = 100000
DIM = 64
N_REFS = 20
BATCH = 16384
MARGIN = 0.5
EPS = 1e-6

NC = 2
NS = 16
NW = NC * NS
CHUNK = BATCH // NW
G = 32
NG = CHUNK // G


IDXC = 128
GR = G * N_REFS
NS_SUB = GR // IDXC
NFLAT = CHUNK * N_REFS


PREFETCH_G = 4


def _sc_embed_body(words_hbm, refs_hbm, win_hbm, wout_hbm,
                   isum, osum, nsum,
                   words_v, pos_v, flat0_v, flat1_v, rows0_v, rows1_v,
                   out0_v, out1_v, sem_r, sem_g0, sem_g1, sem_o0, sem_o1):
  wid = lax.axis_index("s") * NC + lax.axis_index("c")
  base = wid * CHUNK
  rows = (rows0_v, rows1_v)
  flats = (flat0_v, flat1_v)
  outb = (out0_v, out1_v)
  sem_g = (sem_g0, sem_g1)
  sem_o = (sem_o0, sem_o1)

  def stage_ids(a, fa):
    pltpu.sync_copy(words_hbm.at[pl.ds(a * BATCH + wid * CHUNK, CHUNK)],
                    words_v)

    def pos_body(v, _):
      k = v * 16 + lax.iota(jnp.int32, 16)
      row = lax.div(k, jnp.int32(N_REFS))
      col = k - row * N_REFS
      w = plsc.load_gather(words_v, [row])
      pos_v[pl.ds(v * 16, 16)] = w * N_REFS + col
      return 0

    lax.fori_loop(0, NFLAT // 16, pos_body, 0)
    lax.fori_loop(0, NFLAT // IDXC,
                  lambda q, _: (ref_copy(q, fa).start(), 0)[1], 0)

  def ref_copy(q, fa):
    return pltpu.make_async_copy(
        refs_hbm.at[pos_v.at[pl.ds(q * IDXC, IDXC)]],
        flats[fa].at[pl.ds(q * IDXC, IDXC)], sem_r)

  def drain_ids(fa):
    lax.fori_loop(0, NFLAT // IDXC,
                  lambda q, _: (ref_copy(q, fa).wait(), 0)[1], 0)

  stage_ids(0, 0)
  drain_ids(0)

  for a, (table, out) in enumerate(((win_hbm, isum),
                                    (wout_hbm, osum),
                                    (wout_hbm, nsum))):
    fa = a % 2

    def emb_copies(g, b):
      return [
          pltpu.make_async_copy(
              table.at[flats[fa].at[pl.ds(g * GR + s * IDXC, IDXC)]],
              rows[b].at[pl.ds(s * IDXC, IDXC)], sem_g[b])
          for s in range(NS_SUB)
      ]

    def fire(g, b):
      for d in emb_copies(g, b):
        d.start()

    def drain(g, b):
      for d in emb_copies(g, b):
        d.wait()

    def out_copy(g, b):
      return pltpu.make_async_copy(
          outb[b], out.at[pl.ds((base + g * G) // 2, G // 2)], sem_o[b])

    if a > 0:
      drain_ids(fa)
    fire(0, 0)

    def two_groups(h, _):
      for b in range(2):
        g = 2 * h + b

        @pl.when(g + 1 < NG)
        def _():
          fire(g + 1, 1 - b)

        if a < 2:
          @pl.when(g == PREFETCH_G)
          def _():
            stage_ids(a + 1, 1 - fa)

        drain(g, b)

        @pl.when(g >= 2)
        def _():
          out_copy(g - 2, b).wait()

        def item_body(i, _):
          r0 = i * N_REFS
          acc0 = [rows[b][r0, pl.ds(c * 16, 16)] for c in range(DIM // 16)]
          acc1 = [rows[b][r0 + 1, pl.ds(c * 16, 16)]
                  for c in range(DIM // 16)]
          for j in range(2, N_REFS, 2):
            for c in range(DIM // 16):
              acc0[c] = acc0[c] + rows[b][r0 + j, pl.ds(c * 16, 16)]
              acc1[c] = acc1[c] + rows[b][r0 + j + 1, pl.ds(c * 16, 16)]
          half = lax.shift_right_logical(i, 1)
          off = (i & 1) * DIM
          for c in range(DIM // 16):
            outb[b][half, pl.ds(off + c * 16, 16)] = acc0[c] + acc1[c]
          return 0

        lax.fori_loop(0, G, item_body, 0)
        out_copy(g, b).start()
      return 0

    lax.fori_loop(0, NG // 2, two_groups, 0)
    out_copy(NG - 2, 0).wait()
    out_copy(NG - 1, 1).wait()


def _sc_embed(iword, oword, nword, refs, w_in, w_out):
  words = jnp.concatenate([iword, oword, nword])
  refs = refs.reshape(-1)
  mesh = plsc.VectorSubcoreMesh(core_axis_name="c", subcore_axis_name="s")
  f = pl.kernel(
      _sc_embed_body,
      out_type=[jax.ShapeDtypeStruct((BATCH // 2, 2 * DIM), jnp.float32)] * 3,
      mesh=mesh,
      compiler_params=pltpu.CompilerParams(use_tc_tiling_on_sc=False,
                                           needs_layout_passes=False),
      scratch_types=[
          pltpu.VMEM((CHUNK,), jnp.int32),
          pltpu.VMEM((NFLAT,), jnp.int32),
          pltpu.VMEM((NFLAT,), jnp.int32),
          pltpu.VMEM((NFLAT,), jnp.int32),
          pltpu.VMEM((GR, DIM), jnp.float32),
          pltpu.VMEM((GR, DIM), jnp.float32),
          pltpu.VMEM((G // 2, 2 * DIM), jnp.float32),
          pltpu.VMEM((G // 2, 2 * DIM), jnp.float32),
          pltpu.SemaphoreType.DMA,
          pltpu.SemaphoreType.DMA,
          pltpu.SemaphoreType.DMA,
          pltpu.SemaphoreType.DMA,
          pltpu.SemaphoreType.DMA,
      ],
  )
  return f(words, refs, w_in, w_out)


def _log_sigmoid(x):
  return jnp.minimum(x, 0.0) - jnp.log1p(jnp.exp(-jnp.abs(x)))


_ASIN_C = [np.float32(c) for c in (
    1.0, 1 / 6, 3 / 40, 15 / 336, 105 / 3456, 945 / 42240,
    10395 / 599040, 135135 / 9676800)]


def _asin_small(x):
  u = x * x
  s = _ASIN_C[-1]
  for c in _ASIN_C[-2::-1]:
    s = s * u + c
  return x * s


def _half_loss(iv, ov, nv):
  dio = jnp.sum(iv * ov, axis=1)
  din = jnp.sum(iv * nv, axis=1)
  ni = jnp.sqrt(jnp.sum(iv * iv, axis=1))
  no = jnp.sqrt(jnp.sum(ov * ov, axis=1))
  nn = jnp.sqrt(jnp.sum(nv * nv, axis=1))
  cos_io = dio / (jnp.maximum(ni, EPS) * jnp.maximum(no, EPS))
  cos_in = din / (jnp.maximum(ni, EPS) * jnp.maximum(nn, EPS))
  x_p = MARGIN * cos_io
  x_n = MARGIN * cos_in
  pos_angle = np.float32(np.pi / 2) - _asin_small(x_p)
  neg_angle = np.float32(np.pi / 2) - _asin_small(x_n)
  pos_rad = ni * no
  neg_rad = ni * nn
  inv_pi = np.float32(1.0 / np.pi)
  oloss = _log_sigmoid(-pos_angle * pos_rad * inv_pi)
  nloss = _log_sigmoid(neg_angle * neg_rad * inv_pi)
  return jnp.sum(oloss + nloss)


def _loss_body(is_ref, os_ref, ns_ref, out_ref):
  inv = np.float32(1.0 / N_REFS)
  iv = is_ref[...] * inv
  ov = os_ref[...] * inv
  nv = ns_ref[...] * inv
  tot = (_half_loss(iv[:, :DIM], ov[:, :DIM], nv[:, :DIM]) +
         _half_loss(iv[:, DIM:], ov[:, DIM:], nv[:, DIM:]))
  part = -tot * np.float32(1.0 / BATCH)
  pid = pl.program_id(0)

  @pl.when(pid == 0)
  def _():
    out_ref[0, 0] = part

  @pl.when(pid != 0)
  def _():
    out_ref[0, 0] += part


LOSS_BLK = 1024


def _loss(isum, osum, nsum):
  nblk = BATCH // 2 // LOSS_BLK
  spec = pl.BlockSpec((LOSS_BLK, 2 * DIM), lambda i: (i, 0))
  f = pl.pallas_call(
      _loss_body,
      grid=(nblk,),
      in_specs=[spec, spec, spec],
      out_shape=jax.ShapeDtypeStruct((1, 1), jnp.float32),
      out_specs=pl.BlockSpec(memory_space=pltpu.SMEM),
  )
  return f(isum, osum, nsum)[0, 0]


def kernel(iword, oword, nword, refs, W_in, W_out):
  iword = iword.astype(jnp.int32)
  oword = oword.astype(jnp.int32)
  nword = nword.astype(jnp.int32)
  refs = refs.astype(jnp.int32)
  isum, osum, nsum = _sc_embed(iword, oword, nword, refs, W_in, W_out)
  return _loss(isum, osum, nsum)
